# Initial kernel scaffold; baseline (speedup 1.0000x reference)
import jax
import jax.numpy as jnp
from jax.experimental import pallas as pl


def kernel(x, edge_index, edge_attr, batch, W1, b1, g1, be1, W2, b2, g2, be2, W3, b3, g3, be3, Wfc, bfc):
    raise NotImplementedError("write your pallas kernel here")


# speedup vs baseline: 3.7220x; 3.7220x over previous
import functools

import jax
import jax.numpy as jnp
from jax import lax
from jax.experimental import pallas as pl
from jax.experimental.pallas import tpu as pltpu
from jax.experimental.pallas import tpu_sc as plsc

N = 100000
NP = 100352
E = 1600000
EP = 1605632
ER = EP // 128
G = 64
TILN = NP // 16
BT = 2048
GRID = NP // BT

PIECE = 8192
NPIECE = 13
NP13 = NPIECE * PIECE
WROWS = EP // 32 // 128
PWS_ROWS = 408
PWS = PWS_ROWS * 128
EP2 = 32 * PWS
ER2 = EP2 // 128
SW2 = PWS_ROWS // 8
HALF_SW = ER // 2 // 16 // 8

_i32 = jnp.int32
_f32 = jnp.float32


def _iota16():
    return lax.iota(_i32, 16)


def _sc_deg(dst2d, ew1, zeros1, degp, dstb, ewb, acc_sh):
    cid = lax.axis_index("c")
    sid = lax.axis_index("s")
    pltpu.sync_copy(zeros1.at[pl.ds(sid * TILN, TILN)],
                    acc_sh.at[pl.ds(sid * TILN, TILN)])
    plsc.subcore_barrier()

    half = ER // 2
    base = cid * half + sid * (half // 16)

    def body(sw, _):
        row0 = base + sw * 8
        pltpu.sync_copy(dst2d.at[pl.ds(row0, 8)], dstb)
        pltpu.sync_copy(ew1.at[pl.ds(row0 * 128, 1024)], ewb)
        for j in range(8):
            pltpu.sync_copy(ewb.at[pl.ds(j * 128, 128)],
                            acc_sh.at[dstb.at[j]], add=True)
        return _

    lax.fori_loop(0, HALF_SW, body, None)
    plsc.subcore_barrier()
    pltpu.sync_copy(acc_sh.at[pl.ds(sid * TILN, TILN)],
                    degp.at[cid, pl.ds(sid * TILN, TILN)])


def _sc_bucket(src1, dst1, ew1, zeroi, nfill, zerof,
               bsrc1, bdst1, bew1, stbl, wtbl,
               sb, db, eb, posb, tb):
    cid = lax.axis_index("c")
    sid = lax.axis_index("s")
    wid = cid * 16 + sid
    lanes = _iota16()

    def pre(k, _):
        off = (wid * PWS_ROWS + k * 8) * 128
        pltpu.sync_copy(zeroi, bsrc1.at[pl.ds(off, 1024)])
        pltpu.sync_copy(nfill, bdst1.at[pl.ds(off, 1024)])
        pltpu.sync_copy(zerof, bew1.at[pl.ds(off, 1024)])
        return _

    lax.fori_loop(0, SW2, pre, None)

    def count(k, cnt):
        pltpu.sync_copy(dst1.at[pl.ds((wid * WROWS + k) * 128, 128)], db)

        def grp(g, cnt):
            piece = lax.shift_right_logical(db[pl.ds(g * 16, 16)], 13)
            for p in range(NPIECE):
                c = plsc.all_reduce_population_count(piece == p)
                cnt = cnt + jnp.where(lanes == p, c, 0)
            return cnt

        return lax.fori_loop(0, 8, grp, cnt)

    cnt = lax.fori_loop(0, WROWS, count, jnp.zeros((16,), _i32))

    nwin = (cnt + 127) >> 7
    incl = plsc.cumsum(nwin)
    start_rows = wid * PWS_ROWS + (incl - nwin)
    tb[...] = start_rows
    pltpu.sync_copy(tb, stbl.at[pl.ds(wid * 16, 16)])
    tb[...] = nwin
    pltpu.sync_copy(tb, wtbl.at[pl.ds(wid * 16, 16)])

    def scat(k, run):
        base = (wid * WROWS + k) * 128
        pltpu.sync_copy(src1.at[pl.ds(base, 128)], sb)
        pltpu.sync_copy(dst1.at[pl.ds(base, 128)], db)
        pltpu.sync_copy(ew1.at[pl.ds(base, 128)], eb)

        def grp(g, run):
            dv = db[pl.ds(g * 16, 16)]
            piece = lax.shift_right_logical(dv, 13)
            posv = jnp.zeros((16,), _i32)
            for p in range(NPIECE):
                m = piece == p
                c = plsc.cumsum(jnp.where(m, 1, 0))
                posv = jnp.where(m, run[p] + c - 1, posv)
                run = run + jnp.where(lanes == p, c[15], 0)
            posb[pl.ds(g * 16, 16)] = posv
            return run

        run = lax.fori_loop(0, 8, grp, run)
        pltpu.sync_copy(sb, bsrc1.at[posb])
        pltpu.sync_copy(db, bdst1.at[posb])
        pltpu.sync_copy(eb, bew1.at[posb])
        return run

    lax.fori_loop(0, WROWS, scat, start_rows * 128)


def _sc_norm_y1(src2d, dst2d, ew1, dinv1, xp1, zeros1, norm1, y1p,
                srcb, dstb, ewb, dsb, ddb, xsb, nb, vb, sem,
                dinv_sh, x_sh, acc_sh):
    cid = lax.axis_index("c")
    sid = lax.axis_index("s")
    sl = pl.ds(sid * TILN, TILN)

    pltpu.sync_copy(dinv1.at[sl], dinv_sh.at[sl])
    pltpu.sync_copy(xp1.at[sl], x_sh.at[sl])
    pltpu.sync_copy(zeros1.at[sl], acc_sh.at[sl])
    plsc.subcore_barrier()

    base = (cid * 16 + sid) * PWS_ROWS

    def body(sw, _):
        row0 = base + sw * 8
        pltpu.sync_copy(src2d.at[pl.ds(row0, 8)], srcb)
        pltpu.sync_copy(dst2d.at[pl.ds(row0, 8)], dstb)
        pltpu.sync_copy(ew1.at[pl.ds(row0 * 128, 1024)], ewb)
        cps = []
        for j in range(8):
            jj = pl.ds(j * 128, 128)
            cps.append(pltpu.async_copy(dinv_sh.at[srcb.at[j]], dsb.at[jj], sem))
            cps.append(pltpu.async_copy(dinv_sh.at[dstb.at[j]], ddb.at[jj], sem))
            cps.append(pltpu.async_copy(x_sh.at[srcb.at[j]], xsb.at[jj], sem))
        for cp in cps:
            cp.wait()

        def grp(g, _):
            c = pl.ds(g * 16, 16)
            nv = dsb[c] * ewb[c] * ddb[c]
            nb[c] = nv
            vb[c] = xsb[c] * nv
            return _

        lax.fori_loop(0, 64, grp, None)
        pltpu.sync_copy(nb, norm1.at[pl.ds(row0 * 128, 1024)])
        for j in range(8):
            pltpu.sync_copy(vb.at[pl.ds(j * 128, 128)],
                            acc_sh.at[dstb.at[j]], add=True)
        return _

    lax.fori_loop(0, SW2, body, None)
    plsc.subcore_barrier()
    pltpu.sync_copy(acc_sh.at[sl], y1p.at[cid, sl])


def _sc_spmm(bsrc2d, bdst1, bnorm1, hw, zacc, stbl, wtbl, zc,
             srcb, dstb, nb, rb, tvs, tvw, sem, sem2, acc_sh):
    cid = lax.axis_index("c")
    sid = lax.axis_index("s")
    asl = pl.ds(sid * 512, 512)

    pltpu.sync_copy(stbl, tvs)
    pltpu.sync_copy(wtbl, tvw)

    for r in range(7):
        p = cid * 7 + r

        @pl.when(p < NPIECE)
        def _():
            pltpu.sync_copy(zacc.at[asl], acc_sh.at[asl])
            plsc.subcore_barrier()

            for ww in range(2):
                w = sid * 2 + ww
                tix = jnp.full((16,), w * 16 + p, _i32)
                srow = plsc.load_gather(tvs, [tix])[0]
                nw = plsc.load_gather(tvw, [tix])[0]

                def body(k, _):
                    row = srow + k
                    pltpu.sync_copy(bsrc2d.at[pl.ds(row, 1)], srcb)
                    pltpu.sync_copy(bdst1.at[pl.ds(row * 128, 128)], dstb)
                    pltpu.sync_copy(bnorm1.at[pl.ds(row * 128, 128)], nb)
                    pltpu.async_copy(hw.at[srcb.at[0]], rb, sem).wait()

                    def grp(g, _):
                        lo = pl.ds(g * 16, 16)
                        local = dstb[lo] - p * PIECE
                        dstb[lo] = jnp.minimum(
                            jnp.maximum(local, 0), PIECE - 1)
                        for t in range(16):
                            e = g * 16 + t
                            ns = plsc.load_gather(
                                nb, [jnp.full((16,), e, _i32)])
                            for kk in range(8):
                                ck = pl.ds(kk * 16, 16)
                                rb[e, ck] = rb[e, ck] * ns
                        return _

                    lax.fori_loop(0, 8, grp, None)
                    pltpu.async_copy(rb, acc_sh.at[dstb], sem2,
                                     add=True).wait()
                    return _

                lax.fori_loop(0, nw, body, None)

            plsc.subcore_barrier()
            pltpu.sync_copy(acc_sh.at[asl],
                            zc.at[pl.ds(p * PIECE + sid * 512, 512)])
            plsc.subcore_barrier()


def _sc_call(body, out_type, scratch_types):
    return pl.kernel(
        body,
        out_type=out_type,
        mesh=plsc.VectorSubcoreMesh(core_axis_name="c", subcore_axis_name="s"),
        scratch_types=scratch_types,
        compiler_params=pltpu.CompilerParams(needs_layout_passes=False),
    )


def _tc_dinv(degp_ref, dinv_ref):
    i = pl.program_id(0)
    deg = degp_ref[0] + degp_ref[1] + 1.0
    rowid = i * BT + lax.broadcasted_iota(_i32, (BT, 1), 0)
    mask = rowid < N
    dinv = jnp.where(deg > 0, lax.rsqrt(jnp.maximum(deg, 1e-12)), 0.0)
    dinv_ref[...] = jnp.where(mask, dinv, 0.0)


def _tc_y1(y1p_ref, dinv_ref, xp_ref, w1_ref, g1_ref, be1_ref,
           y1_ref, ac_ref, s_ref):
    i = pl.program_id(0)
    d = dinv_ref[...]
    y1 = y1p_ref[0] + y1p_ref[1] + d * d * xp_ref[...]
    rowid = i * BT + lax.broadcasted_iota(_i32, (BT, 1), 0)
    y1 = jnp.where(rowid < N, y1, 0.0)
    y1_ref[...] = y1
    s = jnp.sum(y1)
    ss = jnp.sum(y1 * y1)
    prev_s = jnp.where(i == 0, 0.0, s_ref[0])
    prev_ss = jnp.where(i == 0, 0.0, s_ref[1])
    s_ref[0] = prev_s + s
    s_ref[1] = prev_ss + ss

    @pl.when(i == GRID - 1)
    def _():
        my = s_ref[0] / N
        vy = s_ref[1] / N - my * my
        w1 = w1_ref[...]
        rs = lax.rsqrt(vy * w1 * w1 + 1e-5)
        a = w1 * rs * g1_ref[...]
        c = be1_ref[...] - my * a
        ac_ref[0:1, :] = a
        ac_ref[1:2, :] = c


def _tc_h1(y1_ref, ac_ref, h1w_ref):
    y1 = y1_ref[...]
    h1 = jnp.maximum(y1 * ac_ref[0:1, :] + ac_ref[1:2, :], 0.0)
    h1w_ref[...] = jnp.concatenate([h1, jnp.zeros((BT, 64), _f32)], axis=1)


def _out2_block(z2_ref, y1_ref, dinv_ref, ac_ref, w2_ref, b2_ref):
    y1 = y1_ref[...]
    d = dinv_ref[...]
    h1 = jnp.maximum(y1 * ac_ref[0:1, :] + ac_ref[1:2, :], 0.0)
    zfull = z2_ref[...][:, 0:64] + (d * d) * h1
    return jnp.dot(zfull, w2_ref[...], preferred_element_type=_f32) + b2_ref[...]


def _tc_stats2(z2_ref, y1_ref, dinv_ref, ac_ref, w2_ref, b2_ref, s2_ref):
    i = pl.program_id(0)
    out2 = _out2_block(z2_ref, y1_ref, dinv_ref, ac_ref, w2_ref, b2_ref)
    rowid = i * BT + lax.broadcasted_iota(_i32, (BT, 1), 0)
    out2 = jnp.where(rowid < N, out2, 0.0)
    part = jnp.concatenate([jnp.sum(out2, axis=0, keepdims=True),
                            jnp.sum(out2 * out2, axis=0, keepdims=True)], axis=0)

    @pl.when(i == 0)
    def _():
        s2_ref[...] = jnp.zeros((2, 128), _f32)

    s2_ref[...] += part


def _tc_h2(z2_ref, y1_ref, dinv_ref, ac_ref, w2_ref, b2_ref, s2_ref,
           g2_ref, be2_ref, h2w_ref):
    out2 = _out2_block(z2_ref, y1_ref, dinv_ref, ac_ref, w2_ref, b2_ref)
    m = s2_ref[0:1, :] / N
    v = s2_ref[1:2, :] / N - m * m
    h2w_ref[...] = jnp.maximum(
        (out2 - m) * lax.rsqrt(v + 1e-5) * g2_ref[...] + be2_ref[...], 0.0)


def _out3_block(z3_ref, h2w_ref, dinv_ref, w3_ref, b3_ref):
    d = dinv_ref[...]
    zfull = z3_ref[...] + (d * d) * h2w_ref[...]
    return jnp.dot(zfull, w3_ref[...], preferred_element_type=_f32) + b3_ref[...]


def _tc_stats3(z3_ref, h2w_ref, dinv_ref, w3_ref, b3_ref, s3_ref):
    i = pl.program_id(0)
    out3 = _out3_block(z3_ref, h2w_ref, dinv_ref, w3_ref, b3_ref)
    rowid = i * BT + lax.broadcasted_iota(_i32, (BT, 1), 0)
    out3 = jnp.where(rowid < N, out3, 0.0)
    part = jnp.concatenate([jnp.sum(out3, axis=0, keepdims=True),
                            jnp.sum(out3 * out3, axis=0, keepdims=True)], axis=0)

    @pl.when(i == 0)
    def _():
        s3_ref[...] = jnp.zeros((2, 128), _f32)

    s3_ref[...] += part


def _tc_final(z3_ref, h2w_ref, dinv_ref, s3_ref, g3_ref, be3_ref,
              w3_ref, b3_ref, batch_ref, wfc_ref, bfc_ref,
              res_ref, pooled_ref, counts_ref):
    i = pl.program_id(0)
    out3 = _out3_block(z3_ref, h2w_ref, dinv_ref, w3_ref, b3_ref)
    m = s3_ref[0:1, :] / N
    v = s3_ref[1:2, :] / N - m * m
    h3 = jnp.maximum((out3 - m) * lax.rsqrt(v + 1e-5) * g3_ref[...] + be3_ref[...], 0.0)
    h3 = h3 + h2w_ref[...]
    bb = batch_ref[...]
    gi = lax.broadcasted_iota(_i32, (G, BT), 0)
    M = jnp.where(bb == gi, 1.0, 0.0)

    @pl.when(i == 0)
    def _():
        pooled_ref[...] = jnp.zeros((G, 128), _f32)
        counts_ref[...] = jnp.zeros((G, 1), _f32)

    pooled_ref[...] += jnp.dot(M, h3, preferred_element_type=_f32)
    counts_ref[...] += jnp.sum(M, axis=1, keepdims=True)

    @pl.when(i == GRID - 1)
    def _():
        pool = pooled_ref[...] / jnp.maximum(counts_ref[...], 1.0)
        res_ref[...] = jnp.dot(pool, wfc_ref[...], preferred_element_type=_f32) + bfc_ref[...]


def kernel(x, edge_index, edge_attr, batch, W1, b1, g1, be1, W2, b2, g2, be2,
           W3, b3, g3, be3, Wfc, bfc):
    src = edge_index[0]
    dst = edge_index[1]
    src1 = jnp.pad(src, (0, EP - E)).astype(_i32)
    dst1 = jnp.pad(dst, (0, EP - E), constant_values=N).astype(_i32)
    ew1 = jnp.pad(edge_attr, (0, EP - E))
    xp1 = jnp.pad(x[:, 0], (0, NP - N))
    batch_row = jnp.pad(batch, (0, NP - N), constant_values=2**30).reshape(1, NP).astype(_i32)
    zeros1 = jnp.zeros((NP,), _f32)
    zacc = jnp.zeros((PIECE, 128), _f32)
    zeroi = jnp.zeros((1024,), _i32)
    nfill = jnp.full((1024,), N, _i32)

    f32 = jnp.float32

    bsrc1, bdst1, bew1, stbl, wtbl = _sc_call(
        _sc_bucket,
        out_type=[jax.ShapeDtypeStruct((EP2,), _i32),
                  jax.ShapeDtypeStruct((EP2,), _i32),
                  jax.ShapeDtypeStruct((EP2,), f32),
                  jax.ShapeDtypeStruct((512,), _i32),
                  jax.ShapeDtypeStruct((512,), _i32)],
        scratch_types=[
            pltpu.VMEM((128,), _i32),
            pltpu.VMEM((128,), _i32),
            pltpu.VMEM((128,), f32),
            pltpu.VMEM((128,), _i32),
            pltpu.VMEM((16,), _i32),
        ],
    )(src1, dst1, ew1, zeroi, nfill, zeros1[0:1024])

    degp = _sc_call(
        _sc_deg,
        out_type=jax.ShapeDtypeStruct((2, NP), f32),
        scratch_types=[
            pltpu.VMEM((8, 128), _i32),
            pltpu.VMEM((1024,), f32),
            pltpu.VMEM_SHARED((NP,), f32),
        ],
    )(dst1.reshape(ER, 128), ew1, zeros1)

    dinv = pl.pallas_call(
        _tc_dinv,
        grid=(GRID,),
        in_specs=[pl.BlockSpec((2, BT, 1), lambda i: (0, i, 0))],
        out_specs=pl.BlockSpec((BT, 1), lambda i: (i, 0)),
        out_shape=jax.ShapeDtypeStruct((NP, 1), f32),
    )(degp.reshape(2, NP, 1))

    bnorm1, y1p = _sc_call(
        _sc_norm_y1,
        out_type=[jax.ShapeDtypeStruct((EP2,), f32),
                  jax.ShapeDtypeStruct((2, NP), f32)],
        scratch_types=[
            pltpu.VMEM((8, 128), _i32),
            pltpu.VMEM((8, 128), _i32),
            pltpu.VMEM((1024,), f32),
            pltpu.VMEM((1024,), f32),
            pltpu.VMEM((1024,), f32),
            pltpu.VMEM((1024,), f32),
            pltpu.VMEM((1024,), f32),
            pltpu.VMEM((1024,), f32),
            pltpu.SemaphoreType.DMA,
            pltpu.VMEM_SHARED((NP,), f32),
            pltpu.VMEM_SHARED((NP,), f32),
            pltpu.VMEM_SHARED((NP,), f32),
        ],
    )(bsrc1.reshape(ER2, 128), bdst1.reshape(ER2, 128), bew1,
      dinv[:, 0], xp1, zeros1)

    y1, ac = pl.pallas_call(
        _tc_y1,
        grid=(GRID,),
        in_specs=[
            pl.BlockSpec((2, BT, 1), lambda i: (0, i, 0)),
            pl.BlockSpec((BT, 1), lambda i: (i, 0)),
            pl.BlockSpec((BT, 1), lambda i: (i, 0)),
            pl.BlockSpec((1, 64), lambda i: (0, 0)),
            pl.BlockSpec((1, 64), lambda i: (0, 0)),
            pl.BlockSpec((1, 64), lambda i: (0, 0)),
        ],
        out_specs=[pl.BlockSpec((BT, 1), lambda i: (i, 0)),
                   pl.BlockSpec((2, 64), lambda i: (0, 0))],
        out_shape=[jax.ShapeDtypeStruct((NP, 1), f32),
                   jax.ShapeDtypeStruct((2, 64), f32)],
        scratch_shapes=[pltpu.SMEM((2,), f32)],
    )(y1p.reshape(2, NP, 1), dinv, xp1.reshape(NP, 1),
      W1.reshape(1, 64), g1.reshape(1, 64), be1.reshape(1, 64))

    h1w = pl.pallas_call(
        _tc_h1,
        grid=(GRID,),
        in_specs=[
            pl.BlockSpec((BT, 1), lambda i: (i, 0)),
            pl.BlockSpec((2, 64), lambda i: (0, 0)),
        ],
        out_specs=pl.BlockSpec((BT, 128), lambda i: (i, 0)),
        out_shape=jax.ShapeDtypeStruct((NP, 128), f32),
    )(y1, ac)

    spmm = _sc_call(
        _sc_spmm,
        out_type=jax.ShapeDtypeStruct((NP13, 128), f32),
        scratch_types=[
            pltpu.VMEM((1, 128), _i32),
            pltpu.VMEM((128,), _i32),
            pltpu.VMEM((128,), f32),
            pltpu.VMEM((128, 128), f32),
            pltpu.VMEM((512,), _i32),
            pltpu.VMEM((512,), _i32),
            pltpu.SemaphoreType.DMA,
            pltpu.SemaphoreType.DMA,
            pltpu.VMEM_SHARED((PIECE, 128), f32),
        ],
    )
    bsrc2d = bsrc1.reshape(ER2, 128)

    z2f = spmm(bsrc2d, bdst1, bnorm1, h1w, zacc, stbl, wtbl)

    common_specs = [
        pl.BlockSpec((BT, 128), lambda i: (i, 0)),
        pl.BlockSpec((BT, 1), lambda i: (i, 0)),
        pl.BlockSpec((BT, 1), lambda i: (i, 0)),
        pl.BlockSpec((2, 64), lambda i: (0, 0)),
        pl.BlockSpec((64, 128), lambda i: (0, 0)),
        pl.BlockSpec((1, 128), lambda i: (0, 0)),
    ]
    s2 = pl.pallas_call(
        _tc_stats2,
        grid=(GRID,),
        in_specs=common_specs,
        out_specs=pl.BlockSpec((2, 128), lambda i: (0, 0)),
        out_shape=jax.ShapeDtypeStruct((2, 128), f32),
    )(z2f, y1, dinv, ac, W2, b2.reshape(1, 128))

    h2w = pl.pallas_call(
        _tc_h2,
        grid=(GRID,),
        in_specs=common_specs + [
            pl.BlockSpec((2, 128), lambda i: (0, 0)),
            pl.BlockSpec((1, 128), lambda i: (0, 0)),
            pl.BlockSpec((1, 128), lambda i: (0, 0)),
        ],
        out_specs=pl.BlockSpec((BT, 128), lambda i: (i, 0)),
        out_shape=jax.ShapeDtypeStruct((NP, 128), f32),
    )(z2f, y1, dinv, ac, W2, b2.reshape(1, 128), s2,
      g2.reshape(1, 128), be2.reshape(1, 128))

    z3f = spmm(bsrc2d, bdst1, bnorm1, h2w, zacc, stbl, wtbl)

    specs3 = [
        pl.BlockSpec((BT, 128), lambda i: (i, 0)),
        pl.BlockSpec((BT, 128), lambda i: (i, 0)),
        pl.BlockSpec((BT, 1), lambda i: (i, 0)),
        pl.BlockSpec((128, 128), lambda i: (0, 0)),
        pl.BlockSpec((1, 128), lambda i: (0, 0)),
    ]
    s3 = pl.pallas_call(
        _tc_stats3,
        grid=(GRID,),
        in_specs=specs3,
        out_specs=pl.BlockSpec((2, 128), lambda i: (0, 0)),
        out_shape=jax.ShapeDtypeStruct((2, 128), f32),
    )(z3f, h2w, dinv, W3, b3.reshape(1, 128))

    res = pl.pallas_call(
        _tc_final,
        grid=(GRID,),
        in_specs=[
            pl.BlockSpec((BT, 128), lambda i: (i, 0)),
            pl.BlockSpec((BT, 128), lambda i: (i, 0)),
            pl.BlockSpec((BT, 1), lambda i: (i, 0)),
            pl.BlockSpec((2, 128), lambda i: (0, 0)),
            pl.BlockSpec((1, 128), lambda i: (0, 0)),
            pl.BlockSpec((1, 128), lambda i: (0, 0)),
            pl.BlockSpec((128, 128), lambda i: (0, 0)),
            pl.BlockSpec((1, 128), lambda i: (0, 0)),
            pl.BlockSpec((1, BT), lambda i: (0, i)),
            pl.BlockSpec((128, 1), lambda i: (0, 0)),
            pl.BlockSpec((1, 1), lambda i: (0, 0)),
        ],
        out_specs=pl.BlockSpec((G, 1), lambda i: (0, 0)),
        out_shape=jax.ShapeDtypeStruct((G, 1), f32),
        scratch_shapes=[pltpu.VMEM((G, 128), f32), pltpu.VMEM((G, 1), f32)],
    )(z3f, h2w, dinv, s3, g3.reshape(1, 128), be3.reshape(1, 128),
      W3, b3.reshape(1, 128), batch_row, Wfc, bfc.reshape(1, 1))

    return res
